# baseline (device time: 965836 ns/iter reference)
import jax
import jax.numpy as jnp
from jax import lax
from jax.experimental import pallas as pl
from jax.experimental.pallas import tpu as pltpu

NZ = 4
CE = 192


def kernel(x, assign, W1, W2):
    T, D = x.shape
    EL, _, F = W1.shape
    E = NZ * EL

    p_out = lax.axis_index("z")

    xb = x.astype(jnp.bfloat16)
    w1b = W1.astype(jnp.bfloat16)
    w2b = W2.astype(jnp.bfloat16)

    order = jnp.argsort(assign)
    se = assign[order]
    pos = jnp.arange(T, dtype=jnp.int32) - jnp.searchsorted(
        se, se).astype(jnp.int32)
    slot = se * CE + pos
    sbuf = jnp.zeros((E * CE, D), jnp.bfloat16).at[slot].set(
        xb[order], mode="drop")
    sbuf = jnp.roll(sbuf.reshape(NZ, EL * CE, D), -p_out, axis=0)

    def body(s_ref, w1_ref, w2_ref, res_ref,
             rxb, obuf, w1v, w2v, ld, s1s, s1r, s2s, s2r):
        p = lax.axis_index("z")
        mx = lax.axis_index("x")
        my = lax.axis_index("y")

        lw1 = pltpu.make_async_copy(w1_ref, w1v, ld.at[0])
        lw1.start()
        lw2 = pltpu.make_async_copy(w2_ref, w2v, ld.at[1])
        lw2.start()
        cp = pltpu.make_async_copy(s_ref.at[0], rxb.at[0], ld.at[2])
        cp.start()

        barrier = pltpu.get_barrier_semaphore()
        for k in range(1, NZ):
            pl.semaphore_signal(
                barrier, inc=1,
                device_id=(mx, my, lax.rem(p + k, NZ)),
                device_id_type=pl.DeviceIdType.MESH)
        pl.semaphore_wait(barrier, NZ - 1)

        sends1 = []
        for k in range(1, NZ):
            r = pltpu.make_async_remote_copy(
                src_ref=s_ref.at[k], dst_ref=rxb.at[NZ - k],
                send_sem=s1s.at[k], recv_sem=s1r.at[NZ - k],
                device_id=(mx, my, lax.rem(p + k, NZ)),
                device_id_type=pl.DeviceIdType.MESH)
            r.start()
            sends1.append(r)

        lw1.wait()
        lw2.wait()
        cp.wait()

        def moe(j):
            for e in range(EL):
                xin = rxb[j, e]
                h = jnp.dot(xin, w1v[e],
                            preferred_element_type=jnp.float32)
                hb = jnp.maximum(h, 0.0).astype(jnp.bfloat16)
                o = jnp.dot(hb, w2v[e],
                            preferred_element_type=jnp.float32)
                obuf[j, e] = o.astype(jnp.bfloat16)

        def recv1(j):
            return pltpu.make_async_remote_copy(
                src_ref=s_ref.at[0], dst_ref=rxb.at[j],
                send_sem=s1s.at[j], recv_sem=s1r.at[j],
                device_id=(mx, my, p),
                device_id_type=pl.DeviceIdType.MESH)

        moe(0)
        res_ref[0] = obuf[0]

        sends2 = []
        for j in range(1, NZ):
            recv1(j).wait_recv()
            moe(j)
            r = pltpu.make_async_remote_copy(
                src_ref=obuf.at[j], dst_ref=res_ref.at[NZ - j],
                send_sem=s2s.at[j], recv_sem=s2r.at[NZ - j],
                device_id=(mx, my, lax.rem(p + j, NZ)),
                device_id_type=pl.DeviceIdType.MESH)
            r.start()
            sends2.append(r)

        for r in sends1:
            r.wait_send()
        for r in sends2:
            r.wait_send()
        for m in range(1, NZ):
            pltpu.make_async_remote_copy(
                src_ref=obuf.at[m], dst_ref=res_ref.at[m],
                send_sem=s2s.at[m], recv_sem=s2r.at[m],
                device_id=(mx, my, p),
                device_id_type=pl.DeviceIdType.MESH).wait_recv()

    cparams = pltpu.CompilerParams(
        collective_id=0, vmem_limit_bytes=100 * 1024 * 1024)

    res = pl.pallas_call(
        body,
        out_shape=jax.ShapeDtypeStruct((NZ, EL, CE, D), jnp.bfloat16),
        in_specs=[
            pl.BlockSpec(memory_space=pl.ANY),
            pl.BlockSpec(memory_space=pl.ANY),
            pl.BlockSpec(memory_space=pl.ANY),
        ],
        out_specs=pl.BlockSpec(memory_space=pltpu.VMEM),
        scratch_shapes=[
            pltpu.VMEM((NZ, EL, CE, D), jnp.bfloat16),
            pltpu.VMEM((NZ, EL, CE, D), jnp.bfloat16),
            pltpu.VMEM((EL, D, F), jnp.bfloat16),
            pltpu.VMEM((EL, F, D), jnp.bfloat16),
            pltpu.SemaphoreType.DMA((3,)),
            pltpu.SemaphoreType.DMA((NZ,)),
            pltpu.SemaphoreType.DMA((NZ,)),
            pltpu.SemaphoreType.DMA((NZ,)),
            pltpu.SemaphoreType.DMA((NZ,)),
        ],
        compiler_params=cparams,
    )(sbuf.reshape(NZ, EL, CE, D), w1b, w2b)

    res = jnp.roll(res, p_out, axis=0)
    res_flat = res.reshape(E * CE, D)
    vals = res_flat[slot].astype(jnp.float32)
    out = jnp.zeros((T, D), jnp.float32).at[order].set(vals)
    return out


# device time: 226241 ns/iter; 4.2691x vs baseline; 4.2691x over previous
import jax
import jax.numpy as jnp
from jax import lax
from jax.experimental import pallas as pl
from jax.experimental.pallas import tpu as pltpu

NZ = 4
CE = 192


def kernel(x, assign, W1, W2):
    T, D = x.shape
    EL, _, F = W1.shape
    E = NZ * EL
    B = EL * CE

    xb = x.astype(jnp.bfloat16)
    w1b = W1.astype(jnp.bfloat16)
    w2b = W2.astype(jnp.bfloat16)

    ar = assign.astype(jnp.int32)
    oh = (ar[:, None] == jnp.arange(E, dtype=jnp.int32)[None, :])
    ohi = oh.astype(jnp.int32)
    pos = jnp.sum(ohi * (jnp.cumsum(ohi, axis=0) - ohi), axis=1)
    slot = ar * CE + pos
    slot_r = slot.reshape(1, T)
    slot_c = slot.reshape(T, 1)

    def body(x_ref, sr_ref, sc_ref, w1_ref, w2_ref, out_ref,
             sbuf, rxb, obuf, rcol, w1e, w2e,
             ld, s1s, s1r, s2s, s2r):
        p = lax.axis_index("z")
        mx = lax.axis_index("x")
        my = lax.axis_index("y")

        barrier = pltpu.get_barrier_semaphore()
        for k in range(1, NZ):
            pl.semaphore_signal(
                barrier, inc=1,
                device_id=(mx, my, lax.rem(p + k, NZ)),
                device_id_type=pl.DeviceIdType.MESH)
        pl.semaphore_wait(barrier, NZ - 1)

        xv = x_ref[...]
        srv = sr_ref[...]

        def dispatch(k):
            c = lax.rem(p + k, NZ)
            ii = lax.broadcasted_iota(jnp.int32, (B, T), 0) + c * B
            pt = (ii == srv).astype(jnp.bfloat16)
            sb = jnp.dot(pt, xv, preferred_element_type=jnp.float32)
            sbuf[k] = sb.astype(jnp.bfloat16)

        sends1 = []
        for k in range(1, NZ):
            dispatch(k)
            r = pltpu.make_async_remote_copy(
                src_ref=sbuf.at[k], dst_ref=rxb.at[NZ - k],
                send_sem=s1s.at[k], recv_sem=s1r.at[NZ - k],
                device_id=(mx, my, lax.rem(p + k, NZ)),
                device_id_type=pl.DeviceIdType.MESH)
            r.start()
            sends1.append(r)
        dispatch(0)
        cp = pltpu.make_async_copy(sbuf.at[0], rxb.at[0], ld.at[2])
        cp.start()
        cp.wait()
        for j in range(1, NZ):
            pltpu.make_async_remote_copy(
                src_ref=sbuf.at[0], dst_ref=rxb.at[j],
                send_sem=s1s.at[j], recv_sem=s1r.at[j],
                device_id=(mx, my, p),
                device_id_type=pl.DeviceIdType.MESH).wait_recv()

        for e in range(EL):
            lw1 = pltpu.make_async_copy(w1_ref.at[e], w1e, ld.at[0])
            lw1.start()
            lw2 = pltpu.make_async_copy(w2_ref.at[e], w2e, ld.at[1])
            lw2.start()
            lw1.wait()
            lw2.wait()
            xin = rxb[:, e * CE:(e + 1) * CE, :].reshape(NZ * CE, D)
            h = jnp.dot(xin, w1e[...], preferred_element_type=jnp.float32)
            hb = jnp.maximum(h, 0.0).astype(jnp.bfloat16)
            o = jnp.dot(hb, w2e[...], preferred_element_type=jnp.float32)
            obuf[:, e * CE:(e + 1) * CE, :] = (
                o.astype(jnp.bfloat16).reshape(NZ, CE, D))

        sends2 = []
        for j in range(1, NZ):
            r = pltpu.make_async_remote_copy(
                src_ref=obuf.at[j], dst_ref=rcol.at[NZ - j],
                send_sem=s2s.at[j], recv_sem=s2r.at[NZ - j],
                device_id=(mx, my, lax.rem(p + j, NZ)),
                device_id_type=pl.DeviceIdType.MESH)
            r.start()
            sends2.append(r)

        scv = sc_ref[...]

        def combine(m, rows):
            c = lax.rem(p + m, NZ)
            jj = lax.broadcasted_iota(jnp.int32, (T, B), 1) + c * B
            pm = (scv == jj).astype(jnp.bfloat16)
            return jnp.dot(pm, rows, preferred_element_type=jnp.float32)

        out_ref[...] = combine(0, obuf[0])
        for m in range(1, NZ):
            pltpu.make_async_remote_copy(
                src_ref=obuf.at[0], dst_ref=rcol.at[m],
                send_sem=s2s.at[m], recv_sem=s2r.at[m],
                device_id=(mx, my, p),
                device_id_type=pl.DeviceIdType.MESH).wait_recv()
            out_ref[...] = out_ref[...] + combine(m, rcol[m])

        for r in sends1:
            r.wait_send()
        for r in sends2:
            r.wait_send()

    cparams = pltpu.CompilerParams(
        collective_id=0, vmem_limit_bytes=100 * 1024 * 1024)

    return pl.pallas_call(
        body,
        out_shape=jax.ShapeDtypeStruct((T, D), jnp.float32),
        in_specs=[
            pl.BlockSpec(memory_space=pltpu.VMEM),
            pl.BlockSpec(memory_space=pltpu.VMEM),
            pl.BlockSpec(memory_space=pltpu.VMEM),
            pl.BlockSpec(memory_space=pl.ANY),
            pl.BlockSpec(memory_space=pl.ANY),
        ],
        out_specs=pl.BlockSpec(memory_space=pltpu.VMEM),
        scratch_shapes=[
            pltpu.VMEM((NZ, B, D), jnp.bfloat16),
            pltpu.VMEM((NZ, B, D), jnp.bfloat16),
            pltpu.VMEM((NZ, B, D), jnp.bfloat16),
            pltpu.VMEM((NZ, B, D), jnp.bfloat16),
            pltpu.VMEM((D, F), jnp.bfloat16),
            pltpu.VMEM((F, D), jnp.bfloat16),
            pltpu.SemaphoreType.DMA((3,)),
            pltpu.SemaphoreType.DMA((NZ,)),
            pltpu.SemaphoreType.DMA((NZ,)),
            pltpu.SemaphoreType.DMA((NZ,)),
            pltpu.SemaphoreType.DMA((NZ,)),
        ],
        compiler_params=cparams,
    )(xb, slot_r, slot_c, w1b, w2b)
